# baseline (speedup 1.0000x reference)
import functools

import jax
import jax.numpy as jnp
from jax import lax
from jax.experimental import pallas as pl
from jax.experimental.pallas import tpu as pltpu
from jax.experimental.pallas import tpu_sc as plsc

F32 = jnp.float32

QB = 512
RB = 2048
SB = 256
K = 16


def _knn_body(n, p1_ref, p2t_ref, idx_ref):
    b = pl.program_id(0)
    nt = n // 128
    p1 = p1_ref[0]
    dts = []
    for t in range(nt):
        p2s = p2t_ref[0, :, t * 128:(t + 1) * 128]
        r2 = jnp.sum(p2s * p2s, axis=0, keepdims=True)
        dts.append(r2 - 2.0 * jnp.dot(p1, p2s, preferred_element_type=F32))

    m = dts[0]
    for t in range(1, nt):
        m = jnp.minimum(m, dts[t])

    ciota = lax.broadcasted_iota(jnp.int32, (QB, 128), 1).astype(F32)
    ccols = []
    for _ in range(K):
        mn = jnp.min(m, axis=1, keepdims=True)
        cand = jnp.where(m <= mn, ciota, jnp.float32(256.0))
        cm = jnp.min(cand, axis=1, keepdims=True)
        ccols.append(cm)
        m = jnp.where(cand == cm, jnp.float32(1e30), m)
    chunks = jnp.concatenate(ccols, axis=1).astype(jnp.int32)

    c = jnp.concatenate(
        [jnp.take_along_axis(dts[t], chunks, axis=1) for t in range(nt)],
        axis=1)

    siota = lax.broadcasted_iota(jnp.int32, (QB, nt * K), 1).astype(F32)
    cols = []
    for _ in range(K):
        mn = jnp.min(c, axis=1, keepdims=True)
        cand = jnp.where(c <= mn, siota, jnp.float32(4096.0))
        am = jnp.min(cand, axis=1, keepdims=True)
        cols.append(am)
        c = jnp.where(siota == am, jnp.float32(1e30), c)
    slot = jnp.concatenate(cols, axis=1).astype(jnp.int32)
    t_of = slot // K
    k_of = slot - t_of * K
    cvals = jnp.take_along_axis(chunks, k_of, axis=1)
    idx_ref[0] = cvals * nt + t_of + b * n


def _knn(p1pad, p2t):
    b, n, _ = p1pad.shape
    return pl.pallas_call(
        functools.partial(_knn_body, n),
        grid=(b, n // QB),
        in_specs=[
            pl.BlockSpec((1, QB, 8), lambda i, j: (i, j, 0)),
            pl.BlockSpec((1, 8, n), lambda i, j: (i, 0, 0)),
        ],
        out_specs=pl.BlockSpec((1, QB, K), lambda i, j: (i, j, 0)),
        out_shape=jax.ShapeDtypeStruct((b, n, K), jnp.int32),
    )(p1pad, p2t)


def _gq_body(posA_ref, featA_ref, posQ_ref, featQ_ref,
             wat_ref, wbt_ref, wct_ref, g_ref, q_ref):
    wat = wat_ref[...]
    g_ref[...] = (jnp.dot(posA_ref[...], wat, preferred_element_type=F32)
                  + jnp.dot(featA_ref[...], wbt_ref[...],
                            preferred_element_type=F32))
    q_ref[...] = (jnp.dot(featQ_ref[...], wct_ref[...],
                          preferred_element_type=F32)
                  - jnp.dot(posQ_ref[...], wat, preferred_element_type=F32))


def _gq(posA, featA, posQ, featQ, wat, wbt, wct):
    rows, c = featA.shape
    return pl.pallas_call(
        _gq_body,
        grid=(rows // RB,),
        in_specs=[
            pl.BlockSpec((RB, 8), lambda i: (i, 0)),
            pl.BlockSpec((RB, c), lambda i: (i, 0)),
            pl.BlockSpec((RB, 8), lambda i: (i, 0)),
            pl.BlockSpec((RB, c), lambda i: (i, 0)),
            pl.BlockSpec((8, 128), lambda i: (0, 0)),
            pl.BlockSpec((c, 128), lambda i: (0, 0)),
            pl.BlockSpec((c, 128), lambda i: (0, 0)),
        ],
        out_specs=[
            pl.BlockSpec((RB, 128), lambda i: (i, 0)),
            pl.BlockSpec((RB, 128), lambda i: (i, 0)),
        ],
        out_shape=[
            jax.ShapeDtypeStruct((rows, 128), F32),
            jax.ShapeDtypeStruct((rows, 128), F32),
        ],
    )(posA, featA, posQ, featQ, wat, wbt, wct)


def _sc_gather(table, idx2d, ch):
    info = plsc.get_sparse_core_info()
    nw = info.num_cores * info.num_subcores
    _, per_w = idx2d.shape
    nch = per_w // ch
    grows = nw * per_w
    mesh = plsc.VectorSubcoreMesh(core_axis_name="c", subcore_axis_name="s")

    @functools.partial(
        pl.kernel, mesh=mesh,
        out_type=jax.ShapeDtypeStruct((grows, 128), F32),
        scratch_types=[
            pltpu.VMEM((per_w,), jnp.int32),
            pltpu.VMEM((ch, 128), F32),
            pltpu.VMEM((ch, 128), F32),
            pltpu.SemaphoreType.DMA,
            pltpu.SemaphoreType.DMA,
            pltpu.SemaphoreType.DMA,
            pltpu.SemaphoreType.DMA,
        ],
    )
    def k(table_hbm, idx_hbm, out_hbm, idx_v, rows_a, rows_b, gs_a, gs_b,
          os_a, os_b):
        wid = lax.axis_index("s") * info.num_cores + lax.axis_index("c")
        base = wid * per_w
        pltpu.sync_copy(idx_hbm.at[wid], idx_v)
        bufs = ((rows_a, gs_a, os_a), (rows_b, gs_b, os_b))

        def gather(j, buf, gsem):
            return pltpu.async_copy(
                table_hbm.at[idx_v.at[pl.ds(j * ch, ch)]], buf, gsem)

        outs = [None, None]
        pend = gather(0, rows_a, gs_a)
        for j in range(nch):
            buf, _, osem = bufs[j % 2]
            pend.wait()
            if j + 1 < nch:
                nbuf, ngsem, _ = bufs[(j + 1) % 2]
                if outs[(j + 1) % 2] is not None:
                    outs[(j + 1) % 2].wait()
                pend = gather(j + 1, nbuf, ngsem)
            outs[j % 2] = pltpu.async_copy(
                buf, out_hbm.at[pl.ds(base + j * ch, ch)], osem)
        outs[(nch - 1) % 2].wait()
        if nch >= 2:
            outs[(nch - 2) % 2].wait()

    return k(table, idx2d)


def _stats0_body(t_ref, q_ref, s_ref):
    q3 = jnp.broadcast_to(q_ref[...][:, None, :], (SB, K, 128))
    y = t_ref[...].astype(F32) + q3.reshape(SB * K, 128)
    s = jnp.sum(y, axis=0)
    ss = jnp.sum(y * y, axis=0)

    @pl.when(pl.program_id(0) == 0)
    def _():
        s_ref[...] = jnp.zeros_like(s_ref)

    s_ref[...] += jnp.stack([s, ss])


def _stats0(t, q):
    rows = q.shape[0]
    return pl.pallas_call(
        _stats0_body,
        grid=(rows // SB,),
        in_specs=[
            pl.BlockSpec((SB * K, 128), lambda i: (i, 0)),
            pl.BlockSpec((SB, 128), lambda i: (i, 0)),
        ],
        out_specs=pl.BlockSpec((2, 128), lambda i: (0, 0)),
        out_shape=jax.ShapeDtypeStruct((2, 128), F32),
    )(t, q)


def _form_y0(t_ref, q_ref):
    q3 = jnp.broadcast_to(q_ref[...][:, None, :], (SB, K, 128))
    return t_ref[...].astype(F32) + q3.reshape(SB * K, 128)


def _chain(y0, coeff_refs):
    y = y0
    for a_ref, c_ref, wt_ref in coeff_refs:
        z = jnp.maximum(y * a_ref[...] + c_ref[...], 0.0)
        y = jnp.dot(z.astype(jnp.bfloat16), wt_ref[...].astype(jnp.bfloat16),
                    preferred_element_type=F32)
    return y


def _acc_stats(s_ref, y):
    @pl.when(pl.program_id(0) == 0)
    def _():
        s_ref[...] = jnp.zeros_like(s_ref)

    s_ref[...] += jnp.stack([jnp.sum(y, axis=0), jnp.sum(y * y, axis=0)])


def _coeff_specs(nl):
    out = []
    for _ in range(nl):
        out += [
            pl.BlockSpec((1, 128), lambda i: (0, 0)),
            pl.BlockSpec((1, 128), lambda i: (0, 0)),
            pl.BlockSpec((128, 128), lambda i: (0, 0)),
        ]
    return out


def _chain_stats(t, q, coeffs):
    rows = q.shape[0]
    nl = len(coeffs)

    def body(t_ref, q_ref, *rest):
        crefs = [tuple(rest[3 * i:3 * i + 3]) for i in range(nl)]
        s_ref = rest[3 * nl]
        _acc_stats(s_ref, _chain(_form_y0(t_ref, q_ref), crefs))

    flat = [x for triple in coeffs for x in triple]
    return pl.pallas_call(
        body,
        grid=(rows // SB,),
        in_specs=[
            pl.BlockSpec((SB * K, 128), lambda i: (i, 0)),
            pl.BlockSpec((SB, 128), lambda i: (i, 0)),
        ] + _coeff_specs(nl),
        out_specs=pl.BlockSpec((2, 128), lambda i: (0, 0)),
        out_shape=jax.ShapeDtypeStruct((2, 128), F32),
    )(t, q, *flat)


def _chain_final(t, q, coeffs, alast, clast):
    rows = q.shape[0]
    nl = len(coeffs)

    def body(t_ref, q_ref, *rest):
        crefs = [tuple(rest[3 * i:3 * i + 3]) for i in range(nl)]
        al_ref, cl_ref, o_ref = rest[3 * nl:3 * nl + 3]
        y = _chain(_form_y0(t_ref, q_ref), crefs)
        z = jnp.maximum(y * al_ref[...] + cl_ref[...], 0.0)
        o_ref[...] = jnp.sum(z.reshape(SB, K, 128), axis=1)

    flat = [x for triple in coeffs for x in triple]
    return pl.pallas_call(
        body,
        grid=(rows // SB,),
        in_specs=[
            pl.BlockSpec((SB * K, 128), lambda i: (i, 0)),
            pl.BlockSpec((SB, 128), lambda i: (i, 0)),
        ] + _coeff_specs(nl) + [
            pl.BlockSpec((1, 128), lambda i: (0, 0)),
            pl.BlockSpec((1, 128), lambda i: (0, 0)),
        ],
        out_specs=pl.BlockSpec((SB, 128), lambda i: (i, 0)),
        out_shape=jax.ShapeDtypeStruct((rows, 128), F32),
    )(t, q, *flat, alast, clast)


def _chain_final_gq(t, q, coeffs, alast, clast, posq, featq, wat, wbt, wct):
    rows = q.shape[0]
    nl = len(coeffs)

    def body(t_ref, q_ref, *rest):
        crefs = [tuple(rest[3 * i:3 * i + 3]) for i in range(nl)]
        (al_ref, cl_ref, posq_ref, featq_ref, wat_ref, wbt_ref, wct_ref,
         g_ref, qo_ref) = rest[3 * nl:]
        y = _chain(_form_y0(t_ref, q_ref), crefs)
        z = jnp.maximum(y * al_ref[...] + cl_ref[...], 0.0)
        f = jnp.sum(z.reshape(SB, K, 128), axis=1)
        wat = wat_ref[...]
        g_ref[...] = (jnp.dot(posq_ref[...], wat, preferred_element_type=F32)
                      + jnp.dot(f, wbt_ref[...], preferred_element_type=F32))
        qo_ref[...] = (jnp.dot(featq_ref[...], wct_ref[...],
                               preferred_element_type=F32)
                       - jnp.dot(posq_ref[...], wat,
                                 preferred_element_type=F32))

    flat = [x for triple in coeffs for x in triple]
    return pl.pallas_call(
        body,
        grid=(rows // SB,),
        in_specs=[
            pl.BlockSpec((SB * K, 128), lambda i: (i, 0)),
            pl.BlockSpec((SB, 128), lambda i: (i, 0)),
        ] + _coeff_specs(nl) + [
            pl.BlockSpec((1, 128), lambda i: (0, 0)),
            pl.BlockSpec((1, 128), lambda i: (0, 0)),
            pl.BlockSpec((SB, 8), lambda i: (i, 0)),
            pl.BlockSpec((SB, 128), lambda i: (i, 0)),
            pl.BlockSpec((8, 128), lambda i: (0, 0)),
            pl.BlockSpec((128, 128), lambda i: (0, 0)),
            pl.BlockSpec((128, 128), lambda i: (0, 0)),
        ],
        out_specs=[
            pl.BlockSpec((SB, 128), lambda i: (i, 0)),
            pl.BlockSpec((SB, 128), lambda i: (i, 0)),
        ],
        out_shape=[
            jax.ShapeDtypeStruct((rows, 128), F32),
            jax.ShapeDtypeStruct((rows, 128), F32),
        ],
    )(t, q, *flat, alast, clast, posq, featq, wat, wbt, wct)


def _bn_coeffs(sums, count, g, b):
    mean = sums[0] / count
    var = sums[1] / count - mean * mean
    a = g * lax.rsqrt(var + 1e-5)
    c = b - mean * a
    return a.reshape(1, 128), c.reshape(1, 128)


def kernel(pos1, pos2, feature1, feature2, W1_0, W1_1, W1_2, W2_0, W2_1,
           g1_0, g1_1, g1_2, g2_0, g2_1, b1_0, b1_1, b1_2, b2_0, b2_1):
    B, N, _ = pos1.shape
    C = feature1.shape[-1]
    rows = B * N
    grows = rows * K
    count = jnp.float32(grows)

    info = plsc.get_sparse_core_info()
    nw = info.num_cores * info.num_subcores
    ch = 256
    per_w = grows // nw

    nt = N // 128
    p1pad = jnp.pad(pos1, ((0, 0), (0, 0), (0, 5)))

    def _permt(pos):
        perm = pos.reshape(B, 128, nt, 3).transpose(0, 2, 1, 3).reshape(B, N, 3)
        return jnp.pad(perm, ((0, 0), (0, 0), (0, 5))).transpose(0, 2, 1)

    idx2 = _knn(p1pad, _permt(pos2)).reshape(nw, per_w)
    idx1 = _knn(p1pad, _permt(pos1)).reshape(nw, per_w)

    p1f = p1pad.reshape(rows, 8)
    p2f = jnp.pad(pos2, ((0, 0), (0, 0), (0, 5))).reshape(rows, 8)
    f1f = feature1.reshape(rows, C)
    f2f = feature2.reshape(rows, C)

    wat = jnp.pad(W1_0[:, :3].T, ((0, 5), (0, 0)))
    wbt = W1_0[:, 3:3 + C].T
    wct = W1_0[:, 3 + C:].T
    G2, Q2 = _gq(p2f, f2f, p1f, f2f, wat, wbt, wct)
    T0 = _sc_gather(G2, idx2, ch)
    a0, c0 = _bn_coeffs(_stats0(T0, Q2), count, g1_0, b1_0)
    l0 = (a0, c0, W1_1.T)
    a1, c1 = _bn_coeffs(_chain_stats(T0, Q2, [l0]), count, g1_1, b1_1)
    l1 = (a1, c1, W1_2.T)
    a2, c2 = _bn_coeffs(_chain_stats(T0, Q2, [l0, l1]), count, g1_2, b1_2)

    wat2 = jnp.pad(W2_0[:, :3].T, ((0, 5), (0, 0)))
    wbt2 = W2_0[:, 3:3 + 128].T
    wct2 = W2_0[:, 3 + 128:].T
    G1, Q1 = _chain_final_gq(T0, Q2, [l0, l1], a2, c2,
                             p1f, f1f, wat2, wbt2, wct2)
    T1 = _sc_gather(G1, idx1, ch)
    a3, c3 = _bn_coeffs(_stats0(T1, Q1), count, g2_0, b2_0)
    l3 = (a3, c3, W2_1.T)
    a4, c4 = _bn_coeffs(_chain_stats(T1, Q1, [l3]), count, g2_1, b2_1)
    feat_out = _chain_final(T1, Q1, [l3], a4, c4).reshape(B, N, 128)

    return pos1, feat_out


# speedup vs baseline: 1.0899x; 1.0899x over previous
import functools

import jax
import jax.numpy as jnp
from jax import lax
from jax.experimental import pallas as pl
from jax.experimental.pallas import tpu as pltpu
from jax.experimental.pallas import tpu_sc as plsc

F32 = jnp.float32

QB = 512
RB = 2048
SB = 512
K = 16


def _knn_body(n, p1_ref, p2t_ref, idx_ref):
    b = pl.program_id(0)
    nt = n // 128
    p1 = p1_ref[0]
    dts = []
    for t in range(nt):
        p2s = p2t_ref[0, :, t * 128:(t + 1) * 128]
        r2 = jnp.sum(p2s * p2s, axis=0, keepdims=True)
        dts.append(r2 - 2.0 * jnp.dot(p1, p2s, preferred_element_type=F32))

    m = dts[0]
    for t in range(1, nt):
        m = jnp.minimum(m, dts[t])

    ciota = lax.broadcasted_iota(jnp.int32, (QB, 128), 1).astype(F32)
    ccols = []
    for _ in range(K):
        mn = jnp.min(m, axis=1, keepdims=True)
        cand = jnp.where(m <= mn, ciota, jnp.float32(256.0))
        cm = jnp.min(cand, axis=1, keepdims=True)
        ccols.append(cm)
        m = jnp.where(cand == cm, jnp.float32(1e30), m)
    chunks = jnp.concatenate(ccols, axis=1).astype(jnp.int32)

    c = jnp.concatenate(
        [jnp.take_along_axis(dts[t], chunks, axis=1) for t in range(nt)],
        axis=1)

    siota = lax.broadcasted_iota(jnp.int32, (QB, nt * K), 1).astype(F32)
    cols = []
    for _ in range(K):
        mn = jnp.min(c, axis=1, keepdims=True)
        cand = jnp.where(c <= mn, siota, jnp.float32(4096.0))
        am = jnp.min(cand, axis=1, keepdims=True)
        cols.append(am)
        c = jnp.where(siota == am, jnp.float32(1e30), c)
    slot = jnp.concatenate(cols, axis=1).astype(jnp.int32)
    t_of = slot // K
    k_of = slot - t_of * K
    cvals = jnp.take_along_axis(chunks, k_of, axis=1)
    idx_ref[0] = cvals * nt + t_of + b * n


def _knn(p1pad, p2t):
    b, n, _ = p1pad.shape
    return pl.pallas_call(
        functools.partial(_knn_body, n),
        grid=(b, n // QB),
        in_specs=[
            pl.BlockSpec((1, QB, 8), lambda i, j: (i, j, 0)),
            pl.BlockSpec((1, 8, n), lambda i, j: (i, 0, 0)),
        ],
        out_specs=pl.BlockSpec((1, QB, K), lambda i, j: (i, j, 0)),
        out_shape=jax.ShapeDtypeStruct((b, n, K), jnp.int32),
    )(p1pad, p2t)


def _gq_body(posA_ref, featA_ref, posQ_ref, featQ_ref,
             wat_ref, wbt_ref, wct_ref, g_ref, q_ref):
    wat = wat_ref[...]
    g_ref[...] = (jnp.dot(posA_ref[...], wat, preferred_element_type=F32)
                  + jnp.dot(featA_ref[...], wbt_ref[...],
                            preferred_element_type=F32))
    q_ref[...] = (jnp.dot(featQ_ref[...], wct_ref[...],
                          preferred_element_type=F32)
                  - jnp.dot(posQ_ref[...], wat, preferred_element_type=F32))


def _gq(posA, featA, posQ, featQ, wat, wbt, wct):
    rows, c = featA.shape
    return pl.pallas_call(
        _gq_body,
        grid=(rows // RB,),
        in_specs=[
            pl.BlockSpec((RB, 8), lambda i: (i, 0)),
            pl.BlockSpec((RB, c), lambda i: (i, 0)),
            pl.BlockSpec((RB, 8), lambda i: (i, 0)),
            pl.BlockSpec((RB, c), lambda i: (i, 0)),
            pl.BlockSpec((8, 128), lambda i: (0, 0)),
            pl.BlockSpec((c, 128), lambda i: (0, 0)),
            pl.BlockSpec((c, 128), lambda i: (0, 0)),
        ],
        out_specs=[
            pl.BlockSpec((RB, 128), lambda i: (i, 0)),
            pl.BlockSpec((RB, 128), lambda i: (i, 0)),
        ],
        out_shape=[
            jax.ShapeDtypeStruct((rows, 128), F32),
            jax.ShapeDtypeStruct((rows, 128), F32),
        ],
    )(posA, featA, posQ, featQ, wat, wbt, wct)


def _sc_gather(table, idx2d, ch):
    info = plsc.get_sparse_core_info()
    nw = info.num_cores * info.num_subcores
    _, per_w = idx2d.shape
    nch = per_w // ch
    grows = nw * per_w
    mesh = plsc.VectorSubcoreMesh(core_axis_name="c", subcore_axis_name="s")

    @functools.partial(
        pl.kernel, mesh=mesh,
        out_type=jax.ShapeDtypeStruct((grows, 128), F32),
        scratch_types=[
            pltpu.VMEM((per_w,), jnp.int32),
            pltpu.VMEM((ch, 128), F32),
            pltpu.VMEM((ch, 128), F32),
            pltpu.SemaphoreType.DMA,
            pltpu.SemaphoreType.DMA,
            pltpu.SemaphoreType.DMA,
            pltpu.SemaphoreType.DMA,
        ],
    )
    def k(table_hbm, idx_hbm, out_hbm, idx_v, rows_a, rows_b, gs_a, gs_b,
          os_a, os_b):
        wid = lax.axis_index("s") * info.num_cores + lax.axis_index("c")
        base = wid * per_w
        pltpu.sync_copy(idx_hbm.at[wid], idx_v)
        bufs = ((rows_a, gs_a, os_a), (rows_b, gs_b, os_b))

        def gather(j, buf, gsem):
            return pltpu.async_copy(
                table_hbm.at[idx_v.at[pl.ds(j * ch, ch)]], buf, gsem)

        outs = [None, None]
        pend = gather(0, rows_a, gs_a)
        for j in range(nch):
            buf, _, osem = bufs[j % 2]
            pend.wait()
            if j + 1 < nch:
                nbuf, ngsem, _ = bufs[(j + 1) % 2]
                if outs[(j + 1) % 2] is not None:
                    outs[(j + 1) % 2].wait()
                pend = gather(j + 1, nbuf, ngsem)
            outs[j % 2] = pltpu.async_copy(
                buf, out_hbm.at[pl.ds(base + j * ch, ch)], osem)
        outs[(nch - 1) % 2].wait()
        if nch >= 2:
            outs[(nch - 2) % 2].wait()

    return k(table, idx2d)


def _stats0_body(t_ref, q_ref, s_ref):
    q3 = jnp.broadcast_to(q_ref[...][:, None, :], (SB, K, 128))
    y = t_ref[...].astype(F32) + q3.reshape(SB * K, 128)
    s = jnp.sum(y, axis=0)
    ss = jnp.sum(y * y, axis=0)

    @pl.when(pl.program_id(0) == 0)
    def _():
        s_ref[...] = jnp.zeros_like(s_ref)

    s_ref[...] += jnp.stack([s, ss])


def _stats0(t, q):
    rows = q.shape[0]
    return pl.pallas_call(
        _stats0_body,
        grid=(rows // SB,),
        in_specs=[
            pl.BlockSpec((SB * K, 128), lambda i: (i, 0)),
            pl.BlockSpec((SB, 128), lambda i: (i, 0)),
        ],
        out_specs=pl.BlockSpec((2, 128), lambda i: (0, 0)),
        out_shape=jax.ShapeDtypeStruct((2, 128), F32),
    )(t, q)


def _form_y0(t_ref, q_ref):
    q3 = jnp.broadcast_to(q_ref[...][:, None, :], (SB, K, 128))
    return t_ref[...].astype(F32) + q3.reshape(SB * K, 128)


def _chain(y0, coeff_refs):
    y = y0
    for a_ref, c_ref, wt_ref in coeff_refs:
        z = jnp.maximum(y * a_ref[...] + c_ref[...], 0.0)
        y = jnp.dot(z, wt_ref[...], preferred_element_type=F32)
    return y


def _acc_stats(s_ref, y):
    @pl.when(pl.program_id(0) == 0)
    def _():
        s_ref[...] = jnp.zeros_like(s_ref)

    s_ref[...] += jnp.stack([jnp.sum(y, axis=0), jnp.sum(y * y, axis=0)])


def _coeff_specs(nl):
    out = []
    for _ in range(nl):
        out += [
            pl.BlockSpec((1, 128), lambda i: (0, 0)),
            pl.BlockSpec((1, 128), lambda i: (0, 0)),
            pl.BlockSpec((128, 128), lambda i: (0, 0)),
        ]
    return out


def _chain_stats(t, q, coeffs):
    rows = q.shape[0]
    nl = len(coeffs)

    def body(t_ref, q_ref, *rest):
        crefs = [tuple(rest[3 * i:3 * i + 3]) for i in range(nl)]
        s_ref = rest[3 * nl]
        _acc_stats(s_ref, _chain(_form_y0(t_ref, q_ref), crefs))

    flat = [x for triple in coeffs for x in triple]
    return pl.pallas_call(
        body,
        grid=(rows // SB,),
        in_specs=[
            pl.BlockSpec((SB * K, 128), lambda i: (i, 0)),
            pl.BlockSpec((SB, 128), lambda i: (i, 0)),
        ] + _coeff_specs(nl),
        out_specs=pl.BlockSpec((2, 128), lambda i: (0, 0)),
        out_shape=jax.ShapeDtypeStruct((2, 128), F32),
    )(t, q, *flat)


def _chain_final(t, q, coeffs, alast, clast):
    rows = q.shape[0]
    nl = len(coeffs)

    def body(t_ref, q_ref, *rest):
        crefs = [tuple(rest[3 * i:3 * i + 3]) for i in range(nl)]
        al_ref, cl_ref, o_ref = rest[3 * nl:3 * nl + 3]
        y = _chain(_form_y0(t_ref, q_ref), crefs)
        z = jnp.maximum(y * al_ref[...] + cl_ref[...], 0.0)
        o_ref[...] = jnp.sum(z.reshape(SB, K, 128), axis=1)

    flat = [x for triple in coeffs for x in triple]
    return pl.pallas_call(
        body,
        grid=(rows // SB,),
        in_specs=[
            pl.BlockSpec((SB * K, 128), lambda i: (i, 0)),
            pl.BlockSpec((SB, 128), lambda i: (i, 0)),
        ] + _coeff_specs(nl) + [
            pl.BlockSpec((1, 128), lambda i: (0, 0)),
            pl.BlockSpec((1, 128), lambda i: (0, 0)),
        ],
        out_specs=pl.BlockSpec((SB, 128), lambda i: (i, 0)),
        out_shape=jax.ShapeDtypeStruct((rows, 128), F32),
    )(t, q, *flat, alast, clast)


def _chain_final_gq(t, q, coeffs, alast, clast, posq, featq, wat, wbt, wct):
    rows = q.shape[0]
    nl = len(coeffs)

    def body(t_ref, q_ref, *rest):
        crefs = [tuple(rest[3 * i:3 * i + 3]) for i in range(nl)]
        (al_ref, cl_ref, posq_ref, featq_ref, wat_ref, wbt_ref, wct_ref,
         g_ref, qo_ref) = rest[3 * nl:]
        y = _chain(_form_y0(t_ref, q_ref), crefs)
        z = jnp.maximum(y * al_ref[...] + cl_ref[...], 0.0)
        f = jnp.sum(z.reshape(SB, K, 128), axis=1)
        wat = wat_ref[...]
        g_ref[...] = (jnp.dot(posq_ref[...], wat, preferred_element_type=F32)
                      + jnp.dot(f, wbt_ref[...], preferred_element_type=F32))
        qo_ref[...] = (jnp.dot(featq_ref[...], wct_ref[...],
                               preferred_element_type=F32)
                       - jnp.dot(posq_ref[...], wat,
                                 preferred_element_type=F32))

    flat = [x for triple in coeffs for x in triple]
    return pl.pallas_call(
        body,
        grid=(rows // SB,),
        in_specs=[
            pl.BlockSpec((SB * K, 128), lambda i: (i, 0)),
            pl.BlockSpec((SB, 128), lambda i: (i, 0)),
        ] + _coeff_specs(nl) + [
            pl.BlockSpec((1, 128), lambda i: (0, 0)),
            pl.BlockSpec((1, 128), lambda i: (0, 0)),
            pl.BlockSpec((SB, 8), lambda i: (i, 0)),
            pl.BlockSpec((SB, 128), lambda i: (i, 0)),
            pl.BlockSpec((8, 128), lambda i: (0, 0)),
            pl.BlockSpec((128, 128), lambda i: (0, 0)),
            pl.BlockSpec((128, 128), lambda i: (0, 0)),
        ],
        out_specs=[
            pl.BlockSpec((SB, 128), lambda i: (i, 0)),
            pl.BlockSpec((SB, 128), lambda i: (i, 0)),
        ],
        out_shape=[
            jax.ShapeDtypeStruct((rows, 128), F32),
            jax.ShapeDtypeStruct((rows, 128), F32),
        ],
    )(t, q, *flat, alast, clast, posq, featq, wat, wbt, wct)


def _bn_coeffs(sums, count, g, b):
    mean = sums[0] / count
    var = sums[1] / count - mean * mean
    a = g * lax.rsqrt(var + 1e-5)
    c = b - mean * a
    return a.reshape(1, 128), c.reshape(1, 128)


def kernel(pos1, pos2, feature1, feature2, W1_0, W1_1, W1_2, W2_0, W2_1,
           g1_0, g1_1, g1_2, g2_0, g2_1, b1_0, b1_1, b1_2, b2_0, b2_1):
    B, N, _ = pos1.shape
    C = feature1.shape[-1]
    rows = B * N
    grows = rows * K
    count = jnp.float32(grows)

    info = plsc.get_sparse_core_info()
    nw = info.num_cores * info.num_subcores
    ch = 256
    per_w = grows // nw

    nt = N // 128
    p1pad = jnp.pad(pos1, ((0, 0), (0, 0), (0, 5)))

    def _permt(pos):
        perm = pos.reshape(B, 128, nt, 3).transpose(0, 2, 1, 3).reshape(B, N, 3)
        return jnp.pad(perm, ((0, 0), (0, 0), (0, 5))).transpose(0, 2, 1)

    idx2 = _knn(p1pad, _permt(pos2)).reshape(nw, per_w)
    idx1 = _knn(p1pad, _permt(pos1)).reshape(nw, per_w)

    p1f = p1pad.reshape(rows, 8)
    p2f = jnp.pad(pos2, ((0, 0), (0, 0), (0, 5))).reshape(rows, 8)
    f1f = feature1.reshape(rows, C)
    f2f = feature2.reshape(rows, C)

    wat = jnp.pad(W1_0[:, :3].T, ((0, 5), (0, 0)))
    wbt = W1_0[:, 3:3 + C].T
    wct = W1_0[:, 3 + C:].T
    G2, Q2 = _gq(p2f, f2f, p1f, f2f, wat, wbt, wct)
    T0 = _sc_gather(G2, idx2, ch)
    a0, c0 = _bn_coeffs(_stats0(T0, Q2), count, g1_0, b1_0)
    l0 = (a0, c0, W1_1.T)
    a1, c1 = _bn_coeffs(_chain_stats(T0, Q2, [l0]), count, g1_1, b1_1)
    l1 = (a1, c1, W1_2.T)
    a2, c2 = _bn_coeffs(_chain_stats(T0, Q2, [l0, l1]), count, g1_2, b1_2)

    wat2 = jnp.pad(W2_0[:, :3].T, ((0, 5), (0, 0)))
    wbt2 = W2_0[:, 3:3 + 128].T
    wct2 = W2_0[:, 3 + 128:].T
    G1, Q1 = _chain_final_gq(T0, Q2, [l0, l1], a2, c2,
                             p1f, f1f, wat2, wbt2, wct2)
    T1 = _sc_gather(G1, idx1, ch)
    a3, c3 = _bn_coeffs(_stats0(T1, Q1), count, g2_0, b2_0)
    l3 = (a3, c3, W2_1.T)
    a4, c4 = _bn_coeffs(_chain_stats(T1, Q1, [l3]), count, g2_1, b2_1)
    feat_out = _chain_final(T1, Q1, [l3], a4, c4).reshape(B, N, 128)

    return pos1, feat_out


# speedup vs baseline: 1.1273x; 1.0344x over previous
import functools

import jax
import jax.numpy as jnp
from jax import lax
from jax.experimental import pallas as pl
from jax.experimental.pallas import tpu as pltpu
from jax.experimental.pallas import tpu_sc as plsc

F32 = jnp.float32

QB = 512
RB = 2048
SB = 1024
K = 16


def _knn_body(n, p1_ref, p2t_ref, idx_ref):
    b = pl.program_id(0)
    nt = n // 128
    p1 = p1_ref[0]
    dts = []
    for t in range(nt):
        p2s = p2t_ref[0, :, t * 128:(t + 1) * 128]
        r2 = jnp.sum(p2s * p2s, axis=0, keepdims=True)
        dts.append(r2 - 2.0 * jnp.dot(p1, p2s, preferred_element_type=F32))

    m = dts[0]
    for t in range(1, nt):
        m = jnp.minimum(m, dts[t])

    ciota = lax.broadcasted_iota(jnp.int32, (QB, 128), 1).astype(F32)
    ccols = []
    for _ in range(K):
        mn = jnp.min(m, axis=1, keepdims=True)
        cand = jnp.where(m <= mn, ciota, jnp.float32(256.0))
        cm = jnp.min(cand, axis=1, keepdims=True)
        ccols.append(cm)
        m = jnp.where(cand == cm, jnp.float32(1e30), m)
    chunks = jnp.concatenate(ccols, axis=1).astype(jnp.int32)

    c = jnp.concatenate(
        [jnp.take_along_axis(dts[t], chunks, axis=1) for t in range(nt)],
        axis=1)

    siota = lax.broadcasted_iota(jnp.int32, (QB, nt * K), 1).astype(F32)
    cols = []
    for _ in range(K):
        mn = jnp.min(c, axis=1, keepdims=True)
        cand = jnp.where(c <= mn, siota, jnp.float32(4096.0))
        am = jnp.min(cand, axis=1, keepdims=True)
        cols.append(am)
        c = jnp.where(siota == am, jnp.float32(1e30), c)
    slot = jnp.concatenate(cols, axis=1).astype(jnp.int32)
    t_of = slot // K
    k_of = slot - t_of * K
    cvals = jnp.take_along_axis(chunks, k_of, axis=1)
    idx_ref[0] = cvals * nt + t_of + b * n


def _knn(p1pad, p2t):
    b, n, _ = p1pad.shape
    return pl.pallas_call(
        functools.partial(_knn_body, n),
        grid=(b, n // QB),
        in_specs=[
            pl.BlockSpec((1, QB, 8), lambda i, j: (i, j, 0)),
            pl.BlockSpec((1, 8, n), lambda i, j: (i, 0, 0)),
        ],
        out_specs=pl.BlockSpec((1, QB, K), lambda i, j: (i, j, 0)),
        out_shape=jax.ShapeDtypeStruct((b, n, K), jnp.int32),
    )(p1pad, p2t)


def _gq_body(posA_ref, featA_ref, posQ_ref, featQ_ref,
             wat_ref, wbt_ref, wct_ref, g_ref, q_ref):
    wat = wat_ref[...]
    g_ref[...] = (jnp.dot(posA_ref[...], wat, preferred_element_type=F32)
                  + jnp.dot(featA_ref[...], wbt_ref[...],
                            preferred_element_type=F32))
    q_ref[...] = (jnp.dot(featQ_ref[...], wct_ref[...],
                          preferred_element_type=F32)
                  - jnp.dot(posQ_ref[...], wat, preferred_element_type=F32))


def _gq(posA, featA, posQ, featQ, wat, wbt, wct):
    rows, c = featA.shape
    return pl.pallas_call(
        _gq_body,
        grid=(rows // RB,),
        in_specs=[
            pl.BlockSpec((RB, 8), lambda i: (i, 0)),
            pl.BlockSpec((RB, c), lambda i: (i, 0)),
            pl.BlockSpec((RB, 8), lambda i: (i, 0)),
            pl.BlockSpec((RB, c), lambda i: (i, 0)),
            pl.BlockSpec((8, 128), lambda i: (0, 0)),
            pl.BlockSpec((c, 128), lambda i: (0, 0)),
            pl.BlockSpec((c, 128), lambda i: (0, 0)),
        ],
        out_specs=[
            pl.BlockSpec((RB, 128), lambda i: (i, 0)),
            pl.BlockSpec((RB, 128), lambda i: (i, 0)),
        ],
        out_shape=[
            jax.ShapeDtypeStruct((rows, 128), F32),
            jax.ShapeDtypeStruct((rows, 128), F32),
        ],
    )(posA, featA, posQ, featQ, wat, wbt, wct)


def _sc_gather(table, idx2d, ch):
    info = plsc.get_sparse_core_info()
    nw = info.num_cores * info.num_subcores
    _, per_w = idx2d.shape
    nch = per_w // ch
    grows = nw * per_w
    mesh = plsc.VectorSubcoreMesh(core_axis_name="c", subcore_axis_name="s")

    @functools.partial(
        pl.kernel, mesh=mesh,
        out_type=jax.ShapeDtypeStruct((grows, 128), F32),
        scratch_types=[
            pltpu.VMEM((per_w,), jnp.int32),
            pltpu.VMEM((ch, 128), F32),
            pltpu.VMEM((ch, 128), F32),
            pltpu.SemaphoreType.DMA,
            pltpu.SemaphoreType.DMA,
            pltpu.SemaphoreType.DMA,
            pltpu.SemaphoreType.DMA,
        ],
    )
    def k(table_hbm, idx_hbm, out_hbm, idx_v, rows_a, rows_b, gs_a, gs_b,
          os_a, os_b):
        wid = lax.axis_index("s") * info.num_cores + lax.axis_index("c")
        base = wid * per_w
        pltpu.sync_copy(idx_hbm.at[wid], idx_v)
        bufs = ((rows_a, gs_a, os_a), (rows_b, gs_b, os_b))

        def gather(j, buf, gsem):
            return pltpu.async_copy(
                table_hbm.at[idx_v.at[pl.ds(j * ch, ch)]], buf, gsem)

        outs = [None, None]
        pend = gather(0, rows_a, gs_a)
        for j in range(nch):
            buf, _, osem = bufs[j % 2]
            pend.wait()
            if j + 1 < nch:
                nbuf, ngsem, _ = bufs[(j + 1) % 2]
                if outs[(j + 1) % 2] is not None:
                    outs[(j + 1) % 2].wait()
                pend = gather(j + 1, nbuf, ngsem)
            outs[j % 2] = pltpu.async_copy(
                buf, out_hbm.at[pl.ds(base + j * ch, ch)], osem)
        outs[(nch - 1) % 2].wait()
        if nch >= 2:
            outs[(nch - 2) % 2].wait()

    return k(table, idx2d)


def _stats0_body(t_ref, q_ref, s_ref):
    q3 = jnp.broadcast_to(q_ref[...][:, None, :], (SB, K, 128))
    y = t_ref[...].astype(F32) + q3.reshape(SB * K, 128)
    s = jnp.sum(y, axis=0)
    ss = jnp.sum(y * y, axis=0)

    @pl.when(pl.program_id(0) == 0)
    def _():
        s_ref[...] = jnp.zeros_like(s_ref)

    s_ref[...] += jnp.stack([s, ss])


def _stats0(t, q):
    rows = q.shape[0]
    return pl.pallas_call(
        _stats0_body,
        grid=(rows // SB,),
        in_specs=[
            pl.BlockSpec((SB * K, 128), lambda i: (i, 0)),
            pl.BlockSpec((SB, 128), lambda i: (i, 0)),
        ],
        out_specs=pl.BlockSpec((2, 128), lambda i: (0, 0)),
        out_shape=jax.ShapeDtypeStruct((2, 128), F32),
    )(t, q)


def _form_y0(t_ref, q_ref):
    q3 = jnp.broadcast_to(q_ref[...][:, None, :], (SB, K, 128))
    return t_ref[...].astype(F32) + q3.reshape(SB * K, 128)


def _chain(y0, coeff_refs):
    y = y0
    for a_ref, c_ref, wt_ref in coeff_refs:
        z = jnp.maximum(y * a_ref[...] + c_ref[...], 0.0)
        y = jnp.dot(z, wt_ref[...], preferred_element_type=F32)
    return y


def _acc_stats(s_ref, y):
    @pl.when(pl.program_id(0) == 0)
    def _():
        s_ref[...] = jnp.zeros_like(s_ref)

    s_ref[...] += jnp.stack([jnp.sum(y, axis=0), jnp.sum(y * y, axis=0)])


def _coeff_specs(nl):
    out = []
    for _ in range(nl):
        out += [
            pl.BlockSpec((1, 128), lambda i: (0, 0)),
            pl.BlockSpec((1, 128), lambda i: (0, 0)),
            pl.BlockSpec((128, 128), lambda i: (0, 0)),
        ]
    return out


def _chain_stats(t, q, coeffs):
    rows = q.shape[0]
    nl = len(coeffs)

    def body(t_ref, q_ref, *rest):
        crefs = [tuple(rest[3 * i:3 * i + 3]) for i in range(nl)]
        s_ref = rest[3 * nl]
        _acc_stats(s_ref, _chain(_form_y0(t_ref, q_ref), crefs))

    flat = [x for triple in coeffs for x in triple]
    return pl.pallas_call(
        body,
        grid=(rows // SB,),
        in_specs=[
            pl.BlockSpec((SB * K, 128), lambda i: (i, 0)),
            pl.BlockSpec((SB, 128), lambda i: (i, 0)),
        ] + _coeff_specs(nl),
        out_specs=pl.BlockSpec((2, 128), lambda i: (0, 0)),
        out_shape=jax.ShapeDtypeStruct((2, 128), F32),
    )(t, q, *flat)


def _chain_final(t, q, coeffs, alast, clast):
    rows = q.shape[0]
    nl = len(coeffs)

    def body(t_ref, q_ref, *rest):
        crefs = [tuple(rest[3 * i:3 * i + 3]) for i in range(nl)]
        al_ref, cl_ref, o_ref = rest[3 * nl:3 * nl + 3]
        y = _chain(_form_y0(t_ref, q_ref), crefs)
        z = jnp.maximum(y * al_ref[...] + cl_ref[...], 0.0)
        o_ref[...] = jnp.sum(z.reshape(SB, K, 128), axis=1)

    flat = [x for triple in coeffs for x in triple]
    return pl.pallas_call(
        body,
        grid=(rows // SB,),
        in_specs=[
            pl.BlockSpec((SB * K, 128), lambda i: (i, 0)),
            pl.BlockSpec((SB, 128), lambda i: (i, 0)),
        ] + _coeff_specs(nl) + [
            pl.BlockSpec((1, 128), lambda i: (0, 0)),
            pl.BlockSpec((1, 128), lambda i: (0, 0)),
        ],
        out_specs=pl.BlockSpec((SB, 128), lambda i: (i, 0)),
        out_shape=jax.ShapeDtypeStruct((rows, 128), F32),
    )(t, q, *flat, alast, clast)


def _chain_final_gq(t, q, coeffs, alast, clast, posq, featq, wat, wbt, wct):
    rows = q.shape[0]
    nl = len(coeffs)

    def body(t_ref, q_ref, *rest):
        crefs = [tuple(rest[3 * i:3 * i + 3]) for i in range(nl)]
        (al_ref, cl_ref, posq_ref, featq_ref, wat_ref, wbt_ref, wct_ref,
         g_ref, qo_ref) = rest[3 * nl:]
        y = _chain(_form_y0(t_ref, q_ref), crefs)
        z = jnp.maximum(y * al_ref[...] + cl_ref[...], 0.0)
        f = jnp.sum(z.reshape(SB, K, 128), axis=1)
        wat = wat_ref[...]
        g_ref[...] = (jnp.dot(posq_ref[...], wat, preferred_element_type=F32)
                      + jnp.dot(f, wbt_ref[...], preferred_element_type=F32))
        qo_ref[...] = (jnp.dot(featq_ref[...], wct_ref[...],
                               preferred_element_type=F32)
                       - jnp.dot(posq_ref[...], wat,
                                 preferred_element_type=F32))

    flat = [x for triple in coeffs for x in triple]
    return pl.pallas_call(
        body,
        grid=(rows // SB,),
        in_specs=[
            pl.BlockSpec((SB * K, 128), lambda i: (i, 0)),
            pl.BlockSpec((SB, 128), lambda i: (i, 0)),
        ] + _coeff_specs(nl) + [
            pl.BlockSpec((1, 128), lambda i: (0, 0)),
            pl.BlockSpec((1, 128), lambda i: (0, 0)),
            pl.BlockSpec((SB, 8), lambda i: (i, 0)),
            pl.BlockSpec((SB, 128), lambda i: (i, 0)),
            pl.BlockSpec((8, 128), lambda i: (0, 0)),
            pl.BlockSpec((128, 128), lambda i: (0, 0)),
            pl.BlockSpec((128, 128), lambda i: (0, 0)),
        ],
        out_specs=[
            pl.BlockSpec((SB, 128), lambda i: (i, 0)),
            pl.BlockSpec((SB, 128), lambda i: (i, 0)),
        ],
        out_shape=[
            jax.ShapeDtypeStruct((rows, 128), F32),
            jax.ShapeDtypeStruct((rows, 128), F32),
        ],
    )(t, q, *flat, alast, clast, posq, featq, wat, wbt, wct)


def _bn_coeffs(sums, count, g, b):
    mean = sums[0] / count
    var = sums[1] / count - mean * mean
    a = g * lax.rsqrt(var + 1e-5)
    c = b - mean * a
    return a.reshape(1, 128), c.reshape(1, 128)


def kernel(pos1, pos2, feature1, feature2, W1_0, W1_1, W1_2, W2_0, W2_1,
           g1_0, g1_1, g1_2, g2_0, g2_1, b1_0, b1_1, b1_2, b2_0, b2_1):
    B, N, _ = pos1.shape
    C = feature1.shape[-1]
    rows = B * N
    grows = rows * K
    count = jnp.float32(grows)

    info = plsc.get_sparse_core_info()
    nw = info.num_cores * info.num_subcores
    ch = 256
    per_w = grows // nw

    nt = N // 128
    p1pad = jnp.pad(pos1, ((0, 0), (0, 0), (0, 5)))

    def _permt(pos):
        perm = pos.reshape(B, 128, nt, 3).transpose(0, 2, 1, 3).reshape(B, N, 3)
        return jnp.pad(perm, ((0, 0), (0, 0), (0, 5))).transpose(0, 2, 1)

    idx2 = _knn(p1pad, _permt(pos2)).reshape(nw, per_w)
    idx1 = _knn(p1pad, _permt(pos1)).reshape(nw, per_w)

    p1f = p1pad.reshape(rows, 8)
    p2f = jnp.pad(pos2, ((0, 0), (0, 0), (0, 5))).reshape(rows, 8)
    f1f = feature1.reshape(rows, C)
    f2f = feature2.reshape(rows, C)

    wat = jnp.pad(W1_0[:, :3].T, ((0, 5), (0, 0)))
    wbt = W1_0[:, 3:3 + C].T
    wct = W1_0[:, 3 + C:].T
    G2, Q2 = _gq(p2f, f2f, p1f, f2f, wat, wbt, wct)
    T0 = _sc_gather(G2, idx2, ch)
    a0, c0 = _bn_coeffs(_stats0(T0, Q2), count, g1_0, b1_0)
    l0 = (a0, c0, W1_1.T)
    a1, c1 = _bn_coeffs(_chain_stats(T0, Q2, [l0]), count, g1_1, b1_1)
    l1 = (a1, c1, W1_2.T)
    a2, c2 = _bn_coeffs(_chain_stats(T0, Q2, [l0, l1]), count, g1_2, b1_2)

    wat2 = jnp.pad(W2_0[:, :3].T, ((0, 5), (0, 0)))
    wbt2 = W2_0[:, 3:3 + 128].T
    wct2 = W2_0[:, 3 + 128:].T
    G1, Q1 = _chain_final_gq(T0, Q2, [l0, l1], a2, c2,
                             p1f, f1f, wat2, wbt2, wct2)
    T1 = _sc_gather(G1, idx1, ch)
    a3, c3 = _bn_coeffs(_stats0(T1, Q1), count, g2_0, b2_0)
    l3 = (a3, c3, W2_1.T)
    a4, c4 = _bn_coeffs(_chain_stats(T1, Q1, [l3]), count, g2_1, b2_1)
    feat_out = _chain_final(T1, Q1, [l3], a4, c4).reshape(B, N, 128)

    return pos1, feat_out
